# baseline (speedup 1.0000x reference)
import dataclasses
import functools
import math

import jax
import jax.numpy as jnp
from jax import lax
from jax.experimental import pallas as pl
from jax.experimental.pallas import tpu as pltpu
from jax.experimental.pallas import tpu_sc as plsc

B, C, NNEG, V, D = 16384, 20, 10, 1000000, 64
K = 32
NC, NS = 2, 16
NW = NC * NS
BPW = B // NW
BB = 4
CH = BB * K
NCH = BPW // BB
NPAD = 2 * B


def _sc_scores(inv3, ids3, emb):
    mesh = plsc.VectorSubcoreMesh(core_axis_name="c", subcore_axis_name="s")
    cp = pltpu.CompilerParams()
    if "needs_layout_passes" in pltpu.CompilerParams.__dataclass_fields__:
        cp = dataclasses.replace(cp, needs_layout_passes=False)
    if "use_tc_tiling_on_sc" in pltpu.CompilerParams.__dataclass_fields__:
        cp = dataclasses.replace(cp, use_tc_tiling_on_sc=True)

    @functools.partial(
        pl.kernel,
        mesh=mesh,
        compiler_params=cp,
        out_type=jax.ShapeDtypeStruct((NW, NCH, CH), jnp.float32),
        scratch_types=[
            pltpu.VMEM((NCH, CH), jnp.int32),
            pltpu.VMEM((BPW // 2, 2 * D), jnp.float32),
            pltpu.VMEM((CH, D), jnp.float32),
            pltpu.VMEM((CH, D), jnp.float32),
            pltpu.VMEM((16 * 17,), jnp.float32),
            pltpu.VMEM((NCH, CH), jnp.float32),
            pltpu.SemaphoreType.DMA,
            pltpu.SemaphoreType.DMA,
        ],
    )
    def k(inv_hbm, ids_hbm, emb_hbm, out_hbm,
          ids_v, inv_v, rows0, rows1, tile, scores_v, sem0, sem1):
        wid = lax.axis_index("s") * NC + lax.axis_index("c")
        pltpu.sync_copy(ids_hbm.at[wid], ids_v)
        pltpu.sync_copy(inv_hbm.at[wid], inv_v)

        lanes = lax.iota(jnp.int32, 16)
        lanes17 = lanes * 17
        sign_g1 = jnp.where(lanes < 4, 1.0,
                            jnp.where(lanes < 14, -1.0, 0.0)).astype(jnp.float32)

        def start(c, buf, sem):
            for g in range(CH // 16):
                idv = ids_v[c, pl.ds(g * 16, 16)]
                for r in range(16):
                    kk = g * 16 + r
                    pltpu.make_async_copy(
                        emb_hbm.at[pl.ds(idv[r], 1)],
                        buf.at[pl.ds(kk, 1)], sem).start()

        def wait(buf, sem):
            for kk in range(CH):
                pltpu.make_async_copy(
                    emb_hbm.at[pl.ds(0, 1)], buf.at[pl.ds(kk, 1)], sem).wait()

        def compute(rows, c):
            @pl.loop(0, BB)
            def _(bb):
                b = c * BB + bb
                brow = b // 2
                bcol = (b % 2) * D
                iv0 = inv_v[brow, pl.ds(bcol, 16)]
                iv1 = inv_v[brow, pl.ds(bcol + 16, 16)]
                iv2 = inv_v[brow, pl.ds(bcol + 32, 16)]
                iv3 = inv_v[brow, pl.ds(bcol + 48, 16)]
                base = bb * K
                for g in range(2):
                    for r in range(16):
                        kk = base + g * 16 + r
                        s = rows[kk, pl.ds(0, 16)] * iv0
                        s = s + rows[kk, pl.ds(16, 16)] * iv1
                        s = s + rows[kk, pl.ds(32, 16)] * iv2
                        s = s + rows[kk, pl.ds(48, 16)] * iv3
                        plsc.store_scatter(tile, [lanes17 + r], s)
                    parts = [tile[pl.ds(l * 17, 16)] for l in range(16)]
                    while len(parts) > 1:
                        parts = [parts[i] + parts[i + 1]
                                 for i in range(0, len(parts), 2)]
                    tot = parts[0] * sign_g1 if g == 1 else parts[0]
                    scores_v[c, pl.ds(base + g * 16, 16)] = tot

        start(0, rows0, sem0)
        start(1, rows1, sem1)

        @pl.loop(0, NCH // 2)
        def _(p):
            c0 = 2 * p
            wait(rows0, sem0)
            compute(rows0, c0)

            @pl.when(p + 1 < NCH // 2)
            def _():
                start(c0 + 2, rows0, sem0)

            wait(rows1, sem1)
            compute(rows1, c0 + 1)

            @pl.when(p + 1 < NCH // 2)
            def _():
                start(c0 + 3, rows1, sem1)

        pltpu.sync_copy(scores_v, out_hbm.at[wid])

    return k(inv3, ids3, emb)


def _tc_loss(scores2d):
    def body(s_ref, o_ref):
        x = s_ref[...]
        ls = jnp.minimum(x, 0.0) - jnp.log1p(jnp.exp(-jnp.abs(x)))
        total = jnp.sum(ls) + NPAD * math.log(2.0)
        o_ref[0] = total * (-1.0 / B)

    out = pl.pallas_call(
        body,
        out_shape=jax.ShapeDtypeStruct((1,), jnp.float32),
        out_specs=pl.BlockSpec(memory_space=pltpu.MemorySpace.SMEM),
    )(scores2d)
    return out[0]


def kernel(in_vectors, contexts, neg_contexts, out_emb):
    inv3 = in_vectors.reshape(NW, BPW // 2, 2 * D)
    pad = jnp.zeros((B, K - C - NNEG), jnp.int32)
    ids3 = jnp.concatenate([contexts, neg_contexts, pad], axis=1).reshape(
        NW, NCH, CH)
    scores = _sc_scores(inv3, ids3, out_emb)
    return _tc_loss(scores.reshape(B * K // 128, 128))


# speedup vs baseline: 1.3260x; 1.3260x over previous
import dataclasses
import functools
import math

import jax
import jax.numpy as jnp
from jax import lax
from jax.experimental import pallas as pl
from jax.experimental.pallas import tpu as pltpu
from jax.experimental.pallas import tpu_sc as plsc

B, C, NNEG, V, D = 16384, 20, 10, 1000000, 64
K = 32
NC, NS = 2, 16
NW = NC * NS
BPW = B // NW
BB = 4
CH = BB * K
NCH = BPW // BB
NPAD = 2 * B


def _sc_scores(inv3, ids3, emb):
    mesh = plsc.VectorSubcoreMesh(core_axis_name="c", subcore_axis_name="s")
    cp = pltpu.CompilerParams()
    if "needs_layout_passes" in pltpu.CompilerParams.__dataclass_fields__:
        cp = dataclasses.replace(cp, needs_layout_passes=False)
    if "use_tc_tiling_on_sc" in pltpu.CompilerParams.__dataclass_fields__:
        cp = dataclasses.replace(cp, use_tc_tiling_on_sc=False)

    @functools.partial(
        pl.kernel,
        mesh=mesh,
        compiler_params=cp,
        out_type=jax.ShapeDtypeStruct((NW, NCH, CH), jnp.float32),
        scratch_types=[
            pltpu.VMEM((NCH, 2, CH // 2), jnp.int32),
            pltpu.VMEM((BPW, D), jnp.float32),
            pltpu.VMEM((CH, D), jnp.float32),
            pltpu.VMEM((CH, D), jnp.float32),
            pltpu.VMEM((16, 17), jnp.float32),
            pltpu.VMEM((NCH, CH), jnp.float32),
            pltpu.SemaphoreType.DMA,
            pltpu.SemaphoreType.DMA,
            pltpu.SemaphoreType.DMA,
            pltpu.SemaphoreType.DMA,
        ],
    )
    def k(inv_hbm, ids_hbm, emb_hbm, out_hbm,
          ids_v, inv_v, rows0, rows1, tile, scores_v,
          sem0, sem1, sem2, sem3):
        wid = lax.axis_index("s") * NC + lax.axis_index("c")
        pltpu.sync_copy(ids_hbm.at[wid], ids_v)
        pltpu.sync_copy(inv_hbm.at[wid], inv_v)

        lanes = lax.iota(jnp.int32, 16)
        sign_g1 = jnp.where(lanes < 4, 1.0,
                            jnp.where(lanes < 14, -1.0, 0.0)).astype(jnp.float32)

        H = CH // 2

        def start(c, buf, semi, semr):
            pltpu.make_async_copy(
                emb_hbm.at[ids_v.at[c, 0]], buf.at[pl.ds(0, H)], semi).start()
            for g in range(H // 16):
                idv = ids_v[c, 1, pl.ds(g * 16, 16)]
                for r in range(16):
                    kk = H + g * 16 + r
                    pltpu.make_async_copy(
                        emb_hbm.at[idv[r]], buf.at[kk], semr).start()

        def wait(c, buf, semi, semr):
            pltpu.make_async_copy(
                emb_hbm.at[ids_v.at[c, 0]], buf.at[pl.ds(0, H)], semi).wait()
            for kk in range(H, CH):
                pltpu.make_async_copy(
                    emb_hbm.at[0], buf.at[kk], semr).wait()

        def compute(rows, c):
            @pl.loop(0, BB)
            def _(bb):
                b = c * BB + bb
                iv0 = inv_v[b, pl.ds(0, 16)]
                iv1 = inv_v[b, pl.ds(16, 16)]
                iv2 = inv_v[b, pl.ds(32, 16)]
                iv3 = inv_v[b, pl.ds(48, 16)]
                base = bb * K
                for g in range(2):
                    for r in range(16):
                        kk = base + g * 16 + r
                        s = rows[kk, pl.ds(0, 16)] * iv0
                        s = s + rows[kk, pl.ds(16, 16)] * iv1
                        s = s + rows[kk, pl.ds(32, 16)] * iv2
                        s = s + rows[kk, pl.ds(48, 16)] * iv3
                        plsc.store_scatter(
                            tile, [lanes, jnp.full((16,), r, jnp.int32)], s)
                    parts = [tile[l, pl.ds(0, 16)] for l in range(16)]
                    while len(parts) > 1:
                        parts = [parts[i] + parts[i + 1]
                                 for i in range(0, len(parts), 2)]
                    tot = parts[0] * sign_g1 if g == 1 else parts[0]
                    scores_v[c, pl.ds(base + g * 16, 16)] = tot

        start(0, rows0, sem0, sem1)
        start(1, rows1, sem2, sem3)

        @pl.loop(0, NCH // 2)
        def _(p):
            c0 = 2 * p
            wait(c0, rows0, sem0, sem1)
            compute(rows0, c0)

            @pl.when(p + 1 < NCH // 2)
            def _():
                start(c0 + 2, rows0, sem0, sem1)

            wait(c0 + 1, rows1, sem2, sem3)
            compute(rows1, c0 + 1)

            @pl.when(p + 1 < NCH // 2)
            def _():
                start(c0 + 3, rows1, sem2, sem3)

        pltpu.sync_copy(scores_v, out_hbm.at[wid])

    return k(inv3, ids3, emb)


def _tc_loss(scores2d):
    def body(s_ref, o_ref):
        x = s_ref[...]
        ls = jnp.minimum(x, 0.0) - jnp.log1p(jnp.exp(-jnp.abs(x)))
        total = jnp.sum(ls) + NPAD * math.log(2.0)
        o_ref[0] = total * (-1.0 / B)

    out = pl.pallas_call(
        body,
        out_shape=jax.ShapeDtypeStruct((1,), jnp.float32),
        out_specs=pl.BlockSpec(memory_space=pltpu.MemorySpace.SMEM),
    )(scores2d)
    return out[0]


def kernel(in_vectors, contexts, neg_contexts, out_emb):
    inv3 = in_vectors.reshape(NW, BPW, D)
    pad = jnp.zeros((B, K - C - NNEG), jnp.int32)
    ids3 = jnp.concatenate([contexts, neg_contexts, pad], axis=1).reshape(
        NW, NCH, 2, CH // 2)
    scores = _sc_scores(inv3, ids3, out_emb)
    return _tc_loss(scores.reshape(B * K // 128, 128))


# speedup vs baseline: 1.3807x; 1.0412x over previous
import dataclasses
import functools

import jax
import jax.numpy as jnp
from jax import lax
from jax.experimental import pallas as pl
from jax.experimental.pallas import tpu as pltpu
from jax.experimental.pallas import tpu_sc as plsc

B, C, NNEG, V, D = 16384, 20, 10, 1000000, 64
K = C + NNEG
NPAIR = B * K
NC, NS = 2, 16
NW = NC * NS
PPW = NPAIR // NW
CHS = 128
NCH = PPW // CHS


def _sc_scores(inv2d, sid3, b3, sign3, emb):
    mesh = plsc.VectorSubcoreMesh(core_axis_name="c", subcore_axis_name="s")
    cp = pltpu.CompilerParams()
    if "needs_layout_passes" in pltpu.CompilerParams.__dataclass_fields__:
        cp = dataclasses.replace(cp, needs_layout_passes=False)
    if "use_tc_tiling_on_sc" in pltpu.CompilerParams.__dataclass_fields__:
        cp = dataclasses.replace(cp, use_tc_tiling_on_sc=False)

    @functools.partial(
        pl.kernel,
        mesh=mesh,
        compiler_params=cp,
        out_type=jax.ShapeDtypeStruct((NW, NCH, CHS), jnp.float32),
        scratch_types=[
            pltpu.VMEM((NCH, CHS), jnp.int32),
            pltpu.VMEM((NCH, CHS), jnp.int32),
            pltpu.VMEM((NCH, CHS), jnp.float32),
            pltpu.VMEM((CHS, D), jnp.float32),
            pltpu.VMEM((CHS, D), jnp.float32),
            pltpu.VMEM((CHS, D), jnp.float32),
            pltpu.VMEM((CHS, D), jnp.float32),
            pltpu.VMEM((16, 17), jnp.float32),
            pltpu.VMEM((NCH, CHS), jnp.float32),
            pltpu.SemaphoreType.DMA,
            pltpu.SemaphoreType.DMA,
            pltpu.SemaphoreType.DMA,
            pltpu.SemaphoreType.DMA,
        ],
    )
    def k(inv_hbm, sid_hbm, b_hbm, sign_hbm, emb_hbm, out_hbm,
          sid_v, b_v, sign_v, rowsE0, rowsE1, rowsI0, rowsI1, tile, scores_v,
          semE0, semE1, semI0, semI1):
        wid = lax.axis_index("s") * NC + lax.axis_index("c")
        pltpu.sync_copy(sid_hbm.at[wid], sid_v)
        pltpu.sync_copy(b_hbm.at[wid], b_v)
        pltpu.sync_copy(sign_hbm.at[wid], sign_v)

        lanes = lax.iota(jnp.int32, 16)

        def start(c, bufE, bufI, semE, semI):
            pltpu.make_async_copy(emb_hbm.at[sid_v.at[c]], bufE, semE).start()
            pltpu.make_async_copy(inv_hbm.at[b_v.at[c]], bufI, semI).start()

        def wait(c, bufE, bufI, semE, semI):
            pltpu.make_async_copy(emb_hbm.at[sid_v.at[c]], bufE, semE).wait()
            pltpu.make_async_copy(inv_hbm.at[b_v.at[c]], bufI, semI).wait()

        def compute(rowsE, rowsI, c):
            for g in range(CHS // 16):
                for r in range(16):
                    kk = g * 16 + r
                    s = rowsE[kk, pl.ds(0, 16)] * rowsI[kk, pl.ds(0, 16)]
                    s = s + rowsE[kk, pl.ds(16, 16)] * rowsI[kk, pl.ds(16, 16)]
                    s = s + rowsE[kk, pl.ds(32, 16)] * rowsI[kk, pl.ds(32, 16)]
                    s = s + rowsE[kk, pl.ds(48, 16)] * rowsI[kk, pl.ds(48, 16)]
                    plsc.store_scatter(
                        tile, [lanes, jnp.full((16,), r, jnp.int32)], s)
                parts = [tile[l, pl.ds(0, 16)] for l in range(16)]
                while len(parts) > 1:
                    parts = [parts[i] + parts[i + 1]
                             for i in range(0, len(parts), 2)]
                tot = parts[0] * sign_v[c, pl.ds(g * 16, 16)]
                scores_v[c, pl.ds(g * 16, 16)] = tot

        start(0, rowsE0, rowsI0, semE0, semI0)
        start(1, rowsE1, rowsI1, semE1, semI1)

        @pl.loop(0, NCH // 2)
        def _(p):
            c0 = 2 * p
            wait(c0, rowsE0, rowsI0, semE0, semI0)
            compute(rowsE0, rowsI0, c0)

            @pl.when(p + 1 < NCH // 2)
            def _():
                start(c0 + 2, rowsE0, rowsI0, semE0, semI0)

            wait(c0 + 1, rowsE1, rowsI1, semE1, semI1)
            compute(rowsE1, rowsI1, c0 + 1)

            @pl.when(p + 1 < NCH // 2)
            def _():
                start(c0 + 3, rowsE1, rowsI1, semE1, semI1)

        pltpu.sync_copy(scores_v, out_hbm.at[wid])

    return k(inv2d, sid3, b3, sign3, emb)


def _tc_loss(scores2d):
    def body(s_ref, o_ref):
        x = s_ref[...]
        ls = jnp.minimum(x, 0.0) - jnp.log1p(jnp.exp(-jnp.abs(x)))
        o_ref[0] = jnp.sum(ls) * (-1.0 / B)

    out = pl.pallas_call(
        body,
        out_shape=jax.ShapeDtypeStruct((1,), jnp.float32),
        out_specs=pl.BlockSpec(memory_space=pltpu.MemorySpace.SMEM),
    )(scores2d)
    return out[0]


def kernel(in_vectors, contexts, neg_contexts, out_emb):
    inv2d = in_vectors.reshape(B, D)
    ids_flat = jnp.concatenate([contexts, neg_contexts], axis=1).reshape(-1)
    pair = lax.iota(jnp.int32, NPAIR)
    sid, sp = lax.sort_key_val(ids_flat, pair)
    b = sp // K
    sign = jnp.where(sp % K < C, 1.0, -1.0).astype(jnp.float32)
    scores = _sc_scores(inv2d,
                        sid.reshape(NW, NCH, CHS),
                        b.reshape(NW, NCH, CHS),
                        sign.reshape(NW, NCH, CHS),
                        out_emb)
    return _tc_loss(scores.reshape(NPAIR // 128, 128))
